# baseline (speedup 1.0000x reference)
import functools
import math

import jax
import jax.numpy as jnp
from jax import lax
from jax.experimental import pallas as pl
from jax.experimental.pallas import tpu as pltpu
from jax.experimental.pallas import tpu_sc as plsc

EMB = 64
SCALE = math.sqrt(EMB)
NC = 2
NS = 16
NW = NC * NS
LANES = 16


@functools.lru_cache(maxsize=None)
def _make(batch, seq, vocab):
    assert batch % NW == 0
    rows_per_w = batch // NW

    @functools.partial(
        pl.kernel,
        out_type=jax.ShapeDtypeStruct((batch, seq, EMB), jnp.float32),
        mesh=plsc.VectorSubcoreMesh(
            core_axis_name="c", subcore_axis_name="s",
            num_cores=NC, num_subcores=NS,
        ),
        scratch_types=[
            pltpu.VMEM((rows_per_w, seq), jnp.int32),
            pltpu.VMEM((seq, EMB), jnp.float32),
            pltpu.SemaphoreType.DMA,
        ],
        compiler_params=pltpu.CompilerParams(use_tc_tiling_on_sc=False),
    )
    def emb_kernel(tokens_hbm, table_hbm, out_hbm, idx_v, rows_v, gsem):
        wid = lax.axis_index("s") * NC + lax.axis_index("c")
        base = wid * rows_per_w
        pltpu.sync_copy(tokens_hbm.at[pl.ds(base, rows_per_w)], idx_v)

        @pl.loop(0, rows_per_w)
        def _row(r):
            pltpu.async_copy(
                table_hbm.at[idx_v.at[r]], rows_v, gsem
            ).wait()

            @pl.loop(0, seq, step=4)
            def _scale(i):
                for t in range(4):
                    for j in range(EMB // LANES):
                        sl = (i + t, pl.ds(j * LANES, LANES))
                        rows_v[sl] = rows_v[sl] * SCALE

            pltpu.sync_copy(rows_v, out_hbm.at[base + r])

    return emb_kernel


def kernel(tokens, embedding):
    b, s = tokens.shape
    return _make(b, s, embedding.shape[0])(tokens, embedding)
